# baseline (speedup 1.0000x reference)
import functools
import math

import jax
import jax.numpy as jnp
from jax import lax
from jax.experimental import pallas as pl
from jax.experimental.pallas import tpu as pltpu
from jax.experimental.pallas import tpu_sc as plsc

D_MODEL = 64
D_PAD = 128
SCALE = math.sqrt(D_MODEL)
LANES = 16
NUM_CORES = 2
NUM_SUBCORES = 16
NUM_WORKERS = NUM_CORES * NUM_SUBCORES

B_TOTAL = 4096 * 200
ROWS_PER_WORKER = B_TOTAL // NUM_WORKERS
CHUNK = 160
NCHUNKS = ROWS_PER_WORKER // CHUNK
NBUF = 2


def _emb_kernel(lut_hbm, idx_hbm, out_hbm,
                ih0, ih1, g0, g1, o0, o1, iv0, iv1,
                gs0, gs1, os0, os1, ss0, ss1):
    wid = lax.axis_index("s") * NUM_CORES + lax.axis_index("c")
    base = wid * ROWS_PER_WORKER

    ihs, gbufs, obufs = (ih0, ih1), (g0, g1), (o0, o1)
    isms = (iv0, iv1)
    gsems, osems = (gs0, gs1), (os0, os1)
    ssems = (ss0, ss1)

    def start_idx(k, b):
        pltpu.make_async_copy(
            idx_hbm.at[pl.ds(base + k * CHUNK, CHUNK)], isms[b], ssems[b]
        ).start()

    def wait_idx(b):
        pltpu.make_async_copy(
            idx_hbm.at[pl.ds(base, CHUNK)], isms[b], ssems[b]
        ).wait()
        ism, ihb = isms[b], ihs[b]

        @pl.loop(0, CHUNK, step=LANES)
        def _(r0):
            ihb[pl.ds(r0, LANES)] = (
                lax.shift_right_logical(ism[pl.ds(r0, LANES)], 1)
            )

    def start_gather(b):
        pltpu.make_async_copy(
            lut_hbm.at[ihs[b]], gbufs[b], gsems[b]
        ).start()

    def wait_gather(b):
        pltpu.make_async_copy(
            lut_hbm.at[ihs[b]], gbufs[b], gsems[b]
        ).wait()

    def start_out(k, b):
        pltpu.make_async_copy(
            obufs[b], out_hbm.at[pl.ds(base + k * CHUNK, CHUNK)], osems[b]
        ).start()

    def wait_out(b):
        pltpu.make_async_copy(
            obufs[b], out_hbm.at[pl.ds(base, CHUNK)], osems[b]
        ).wait()

    for b in range(NBUF):
        start_idx(b, b)
    for b in range(NBUF):
        wait_idx(b)
        start_gather(b)

    @pl.loop(0, NCHUNKS, step=NBUF)
    def _(k0):
        for b in range(NBUF):
            k = k0 + b
            wait_gather(b)

            @pl.when(k0 > 0)
            def _():
                wait_out(b)

            gb, ob, ism = gbufs[b], obufs[b], isms[b]

            @plsc.parallel_loop(0, CHUNK, step=LANES)
            def _(r0):
                offs = (ism[pl.ds(r0, LANES)] & 1) * D_MODEL
                for j in range(LANES):
                    half = offs[j]
                    for c in range(D_MODEL // LANES):
                        ob[r0 + j, pl.ds(c * LANES, LANES)] = (
                            gb[r0 + j, pl.ds(half + c * LANES, LANES)] * SCALE
                        )

            start_out(k, b)

            @pl.when(k + NBUF < NCHUNKS)
            def _():
                start_idx(k + NBUF, b)
                wait_idx(b)
                start_gather(b)

    for b in range(NBUF):
        wait_out(b)


@jax.jit
def kernel(x, lut):
    idx = x.reshape(B_TOTAL)
    lutp = jnp.concatenate([lut[0::2], lut[1::2]], axis=1)
    mesh = plsc.VectorSubcoreMesh(core_axis_name="c", subcore_axis_name="s")
    run = pl.kernel(
        _emb_kernel,
        out_type=jax.ShapeDtypeStruct((B_TOTAL, D_MODEL), jnp.float32),
        mesh=mesh,
        scratch_types=[
            pltpu.VMEM((CHUNK,), jnp.int32),
            pltpu.VMEM((CHUNK,), jnp.int32),
            pltpu.VMEM((CHUNK, D_PAD), jnp.float32),
            pltpu.VMEM((CHUNK, D_PAD), jnp.float32),
            pltpu.VMEM((CHUNK, D_MODEL), jnp.float32),
            pltpu.VMEM((CHUNK, D_MODEL), jnp.float32),
            pltpu.VMEM((CHUNK,), jnp.int32),
            pltpu.VMEM((CHUNK,), jnp.int32),
            pltpu.SemaphoreType.DMA,
            pltpu.SemaphoreType.DMA,
            pltpu.SemaphoreType.DMA,
            pltpu.SemaphoreType.DMA,
            pltpu.SemaphoreType.DMA,
            pltpu.SemaphoreType.DMA,
        ],
    )
    out = run(lutp, idx)
    return out.reshape(x.shape[0], x.shape[1], D_MODEL)


# speedup vs baseline: 8.0672x; 8.0672x over previous
import functools
import math

import jax
import jax.numpy as jnp
from jax import lax
from jax.experimental import pallas as pl
from jax.experimental.pallas import tpu as pltpu
from jax.experimental.pallas import tpu_sc as plsc

D_MODEL = 64
D_PAD = 128
SCALE = math.sqrt(D_MODEL)
LANES = 16
NUM_CORES = 2
NUM_SUBCORES = 16
NUM_WORKERS = NUM_CORES * NUM_SUBCORES

B_TOTAL = 4096 * 200
ROWS_PER_WORKER = B_TOTAL // NUM_WORKERS
CHUNK = 160
NCHUNKS = ROWS_PER_WORKER // CHUNK
NBUF = 2


def _emb_kernel(lut_hbm, idx_hbm, out_hbm,
                ih0, ih1, g0, g1, o0, o1, iv0, iv1,
                gs0, gs1, os0, os1, ss0, ss1):
    wid = lax.axis_index("s") * NUM_CORES + lax.axis_index("c")
    base = wid * ROWS_PER_WORKER

    ihs, gbufs, obufs = (ih0, ih1), (g0, g1), (o0, o1)
    isms = (iv0, iv1)
    gsems, osems = (gs0, gs1), (os0, os1)
    ssems = (ss0, ss1)

    def start_idx(k, b):
        pltpu.make_async_copy(
            idx_hbm.at[pl.ds(base + k * CHUNK, CHUNK)], isms[b], ssems[b]
        ).start()

    def wait_idx(b):
        pltpu.make_async_copy(
            idx_hbm.at[pl.ds(base, CHUNK)], isms[b], ssems[b]
        ).wait()
        ism, ihb = isms[b], ihs[b]

        @pl.loop(0, CHUNK, step=LANES)
        def _(r0):
            ihb[pl.ds(r0, LANES)] = (
                lax.shift_right_logical(ism[pl.ds(r0, LANES)], 1)
            )

    def start_gather(b):
        pltpu.make_async_copy(
            lut_hbm.at[ihs[b]], gbufs[b], gsems[b]
        ).start()

    def wait_gather(b):
        pltpu.make_async_copy(
            lut_hbm.at[ihs[b]], gbufs[b], gsems[b]
        ).wait()

    def start_out(k, b):
        pltpu.make_async_copy(
            obufs[b], out_hbm.at[pl.ds(base + k * CHUNK, CHUNK)], osems[b]
        ).start()

    def wait_out(b):
        pltpu.make_async_copy(
            obufs[b], out_hbm.at[pl.ds(base, CHUNK)], osems[b]
        ).wait()

    for b in range(NBUF):
        start_idx(b, b)
    for b in range(NBUF):
        wait_idx(b)
        start_gather(b)

    @pl.loop(0, NCHUNKS, step=NBUF)
    def _(k0):
        for b in range(NBUF):
            k = k0 + b
            wait_gather(b)

            @pl.when(k0 > 0)
            def _():
                wait_out(b)

            gb, ob, ism = gbufs[b], obufs[b], isms[b]

            @plsc.parallel_loop(0, CHUNK, step=LANES)
            def _(r0):
                offs = (ism[pl.ds(r0, LANES)] & 1) * D_MODEL
                for j in range(LANES):
                    half = offs[j]
                    for c in range(D_MODEL // LANES):
                        ob[r0 + j, pl.ds(c * LANES, LANES)] = (
                            gb[r0 + j, pl.ds(half + c * LANES, LANES)] * SCALE
                        )

            start_out(k, b)

            @pl.when(k + NBUF < NCHUNKS)
            def _():
                start_idx(k + NBUF, b)
                wait_idx(b)
                start_gather(b)

    for b in range(NBUF):
        wait_out(b)


@jax.jit
def kernel(x, lut):
    idx = x.reshape(B_TOTAL)
    lutp = lut.reshape(D_MODEL * 1000000 // D_PAD, D_PAD)
    mesh = plsc.VectorSubcoreMesh(core_axis_name="c", subcore_axis_name="s")
    run = pl.kernel(
        _emb_kernel,
        out_type=jax.ShapeDtypeStruct((B_TOTAL, D_MODEL), jnp.float32),
        mesh=mesh,
        scratch_types=[
            pltpu.VMEM((CHUNK,), jnp.int32),
            pltpu.VMEM((CHUNK,), jnp.int32),
            pltpu.VMEM((CHUNK, D_PAD), jnp.float32),
            pltpu.VMEM((CHUNK, D_PAD), jnp.float32),
            pltpu.VMEM((CHUNK, D_MODEL), jnp.float32),
            pltpu.VMEM((CHUNK, D_MODEL), jnp.float32),
            pltpu.VMEM((CHUNK,), jnp.int32),
            pltpu.VMEM((CHUNK,), jnp.int32),
            pltpu.SemaphoreType.DMA,
            pltpu.SemaphoreType.DMA,
            pltpu.SemaphoreType.DMA,
            pltpu.SemaphoreType.DMA,
            pltpu.SemaphoreType.DMA,
            pltpu.SemaphoreType.DMA,
        ],
    )
    out = run(lutp, idx)
    return out.reshape(x.shape[0], x.shape[1], D_MODEL)
